# baseline (speedup 1.0000x reference)
import math

import jax
import jax.numpy as jnp
from jax import lax
from jax.experimental import pallas as pl
from jax.experimental.pallas import tpu as pltpu
from jax.experimental.pallas import tpu_sc as plsc

C = 69
H = 121
Wd = 240
NG = H * Wd
NM = 40962
N = NG + NM
HID = 256
OUT_CH = 69
FACTOR = 4
D = 128
NPAD = 70016
BN = 512

NC, NS, LANES = 2, 16, 16
NW = NC * NS
EP = 303104
EPT = EP // NW
ECH = EPT // 128
MGQ = 5376
MG = NW * MGQ
NMG = 4 * NM

R1_NPASS, R1_SIZE = 5, 7680
R2_NPASS, R2_SIZE = 3, 6912

_SQRT2 = math.sqrt(2.0)
_sc_mesh = plsc.VectorSubcoreMesh(core_axis_name="c", subcore_axis_name="s",
                                  num_cores=NC, num_subcores=NS)


def _gelu(x):
    return x * 0.5 * (1.0 + jax.lax.erf(x / _SQRT2))


def _fill_zero(buf, nrows):
    z16 = jnp.zeros((LANES,), jnp.float32)

    def zr(i, _):
        for j in range(D // LANES):
            buf[i, pl.ds(j * LANES, LANES)] = z16
        return 0
    lax.fori_loop(0, nrows, zr, 0)


def _sc_tap_body(idxf_ref, xrow_ref, g_ref,
                 idx0, idx1, rows0, rows1, g0, g1, w0, w1):
    c = lax.axis_index("c")
    s = lax.axis_index("s")
    wid = s * NC + c
    npair = MGQ // 256

    def mg(u, _):
        off0 = wid * MGQ + (2 * u) * 128
        off1 = off0 + 128

        @pl.when(u > 0)
        def _wait_prev():
            po0 = off0 - 256
            pltpu.make_async_copy(rows0, g_ref.at[pl.ds(po0, 128)], w0).wait()
            pltpu.make_async_copy(rows1, g_ref.at[pl.ds(po0 + 128, 128)], w1).wait()

        pltpu.sync_copy(idxf_ref.at[pl.ds(off0, 128)], idx0.at[0])
        pltpu.sync_copy(idxf_ref.at[pl.ds(off1, 128)], idx1.at[0])
        cg0 = pltpu.async_copy(xrow_ref.at[idx0.at[0]], rows0, g0)
        cg1 = pltpu.async_copy(xrow_ref.at[idx1.at[0]], rows1, g1)
        cg0.wait()
        pltpu.async_copy(rows0, g_ref.at[pl.ds(off0, 128)], w0)
        cg1.wait()
        pltpu.async_copy(rows1, g_ref.at[pl.ds(off1, 128)], w1)
        return 0
    lax.fori_loop(0, npair, mg, 0)
    lo = wid * MGQ + (npair - 1) * 256
    pltpu.make_async_copy(rows0, g_ref.at[pl.ds(lo, 128)], w0).wait()
    pltpu.make_async_copy(rows1, g_ref.at[pl.ds(lo + 128, 128)], w1).wait()


_sc_tap = pl.kernel(
    _sc_tap_body,
    out_type=jax.ShapeDtypeStruct((MG, D), jnp.float32),
    mesh=_sc_mesh,
    scratch_types=[
        pltpu.VMEM((1, 128), jnp.int32),
        pltpu.VMEM((1, 128), jnp.int32),
        pltpu.VMEM((128, D), jnp.float32),
        pltpu.VMEM((128, D), jnp.float32),
        pltpu.SemaphoreType.DMA,
        pltpu.SemaphoreType.DMA,
        pltpu.SemaphoreType.DMA,
        pltpu.SemaphoreType.DMA,
    ],
)


def _make_agg(ntab, npass, rsize, base):
    rpt = rsize // NS
    acc_rows = rsize + 128

    def body(*refs):
        srcp_ref, dstp_ref = refs[0], refs[1]
        tabs = refs[2:2 + ntab]
        outs = refs[2 + ntab:2 + 2 * ntab] if ntab else (refs[2],)
        sc = refs[2 + 2 * ntab:] if ntab else refs[3:]
        (srcbuf, dstbuf, sidx0, sidx1, didx0, didx1,
         rows0, rows1, zbuf, acc, sg0, sg1, ss0, ss1) = sc

        c = lax.axis_index("c")
        s = lax.axis_index("s")
        wid = s * NC + c
        pltpu.sync_copy(srcp_ref.at[pl.ds(wid * EPT, EPT)], srcbuf)
        pltpu.sync_copy(dstp_ref.at[pl.ds(wid * EPT, EPT)], dstbuf)
        _fill_zero(zbuf, 128)
        if not ntab:
            one16 = jnp.ones((LANES,), jnp.float32)

            def fr(i, _):
                for j in range(D // LANES):
                    rows0[i, pl.ds(j * LANES, LANES)] = one16
                    rows1[i, pl.ds(j * LANES, LANES)] = one16
                return 0
            lax.fori_loop(0, 128, fr, 0)

        nz = acc_rows // 128
        nmy = (nz - s + NS - 1) // NS

        def build_idx(t, sidx, didx, lo):
            for j in range(128 // LANES):
                d = dstbuf[pl.ds(t * 128 + j * LANES, LANES)]
                ld = d - lo
                inb = (ld >= 0) & (ld < rsize)
                didx[0, pl.ds(j * LANES, LANES)] = jnp.where(inb, ld, rsize)
                if ntab:
                    sv = srcbuf[pl.ds(t * 128 + j * LANES, LANES)]
                    sidx[0, pl.ds(j * LANES, LANES)] = sv

        for h in range(max(ntab, 1)):
            out_ref = outs[h]
            for r in range(NC * npass):
                lo = base + r * rsize

                def za(i, _):
                    k = s + i * NS
                    pltpu.sync_copy(zbuf, acc.at[pl.ds(k * 128, 128)])
                    return 0
                lax.fori_loop(0, nmy, za, 0)
                plsc.subcore_barrier()

                def pair(u, _):
                    @pl.when(u > 0)
                    def _wait_prev():
                        pltpu.make_async_copy(rows0, acc.at[didx0.at[0]], ss0).wait()
                        pltpu.make_async_copy(rows1, acc.at[didx1.at[0]], ss1).wait()

                    build_idx(2 * u, sidx0, didx0, lo)
                    build_idx(2 * u + 1, sidx1, didx1, lo)
                    if ntab:
                        cg0 = pltpu.async_copy(tabs[h].at[sidx0.at[0]], rows0, sg0)
                        cg1 = pltpu.async_copy(tabs[h].at[sidx1.at[0]], rows1, sg1)
                        cg0.wait()
                        pltpu.async_copy(rows0, acc.at[didx0.at[0]], ss0, add=True)
                        cg1.wait()
                        pltpu.async_copy(rows1, acc.at[didx1.at[0]], ss1, add=True)
                    else:
                        pltpu.async_copy(rows0, acc.at[didx0.at[0]], ss0, add=True)
                        pltpu.async_copy(rows1, acc.at[didx1.at[0]], ss1, add=True)
                    return 0
                lax.fori_loop(0, ECH // 2, pair, 0)
                pltpu.make_async_copy(rows0, acc.at[didx0.at[0]], ss0).wait()
                pltpu.make_async_copy(rows1, acc.at[didx1.at[0]], ss1).wait()

                plsc.subcore_barrier()
                pltpu.sync_copy(acc.at[pl.ds(s * rpt, rpt)],
                                out_ref.at[c, pl.ds(r * rsize + s * rpt, rpt)])
                plsc.subcore_barrier()

    out_rows = NC * npass * rsize
    out1 = jax.ShapeDtypeStruct((NC, out_rows, D), jnp.float32)
    return pl.kernel(
        body,
        out_type=tuple([out1] * ntab) if ntab else out1,
        mesh=_sc_mesh,
        scratch_types=[
            pltpu.VMEM((EPT,), jnp.int32),
            pltpu.VMEM((EPT,), jnp.int32),
            pltpu.VMEM((1, 128), jnp.int32),
            pltpu.VMEM((1, 128), jnp.int32),
            pltpu.VMEM((1, 128), jnp.int32),
            pltpu.VMEM((1, 128), jnp.int32),
            pltpu.VMEM((128, D), jnp.float32),
            pltpu.VMEM((128, D), jnp.float32),
            pltpu.VMEM((128, D), jnp.float32),
            pltpu.VMEM_SHARED((acc_rows, D), jnp.float32),
            pltpu.SemaphoreType.DMA,
            pltpu.SemaphoreType.DMA,
            pltpu.SemaphoreType.DMA,
            pltpu.SemaphoreType.DMA,
        ],
    )


_sc_deg = _make_agg(0, R1_NPASS, R1_SIZE, 0)
_sc_agg1 = _make_agg(1, R1_NPASS, R1_SIZE, 0)
_sc_agg2 = _make_agg(2, R2_NPASS, R2_SIZE, NG)


def _combine_dec_body(lw_ref, lw1_ref, lb_ref, lb1_ref, lc_ref, lbc_ref):
    lc_ref[...] = jnp.dot(lw_ref[...], lw1_ref[...],
                          preferred_element_type=jnp.float32)
    lbc_ref[...] = jnp.dot(lb_ref[...], lw1_ref[...],
                           preferred_element_type=jnp.float32) + lb1_ref[...]


def _scale_grid_body(xrow_ref, deg0_ref, deg1_ref, xs_ref, dis_ref):
    dis = jax.lax.rsqrt(deg0_ref[...][:, :1] + deg1_ref[...][:, :1] + 1.0)
    dis_ref[...] = dis
    xs_ref[...] = xrow_ref[...] * dis


def _mesh_assemble_body(g_ref, w4_ref, deg0_ref, deg1_ref, xs_ref, dis_ref):
    g = g_ref[...]
    w = w4_ref[...]
    mesh = (g[:, 0 * D:1 * D] * w[:, 0:1] + g[:, 1 * D:2 * D] * w[:, 1:2]
            + g[:, 2 * D:3 * D] * w[:, 2:3] + g[:, 3 * D:4 * D] * w[:, 3:4])
    dis = jax.lax.rsqrt(deg0_ref[...][:, :1] + deg1_ref[...][:, :1] + 1.0)
    dis_ref[...] = dis
    xs_ref[...] = mesh * dis


def _gcn1_body(a10_ref, a11_ref, xs_ref, dis_ref, w1_ref, b1_ref, ga_ref, gb_ref):
    out1 = dis_ref[...] * (a10_ref[...] + a11_ref[...] + xs_ref[...])
    h = jnp.dot(out1, w1_ref[...], preferred_element_type=jnp.float32) + b1_ref[...]
    gs = _gelu(h) * dis_ref[...]
    ga_ref[...] = gs[:, :D]
    gb_ref[...] = gs[:, D:]


def _dec_body(a2a0_ref, a2a1_ref, a2b0_ref, a2b1_ref, gsa_ref, gsb_ref, dism_ref,
              w2_ref, b2_ref, lc_ref, lbc_ref, o_ref):
    dism = dism_ref[...]
    out2a = dism * (a2a0_ref[...] + a2a1_ref[...] + gsa_ref[...])
    out2b = dism * (a2b0_ref[...] + a2b1_ref[...] + gsb_ref[...])
    h2 = (jnp.dot(out2a, w2_ref[...][:D], preferred_element_type=jnp.float32)
          + jnp.dot(out2b, w2_ref[...][D:], preferred_element_type=jnp.float32)
          + b2_ref[...])
    o = jnp.dot(h2, lc_ref[...], preferred_element_type=jnp.float32) + lbc_ref[...]
    o_ref[...] = _gelu(o)


def _row_blocks(n_rows, width):
    grid = (pl.cdiv(n_rows, BN),)
    blk = pl.BlockSpec((BN, width), lambda i: (i, 0))
    return grid, blk


def _full(shape):
    return pl.BlockSpec(shape, lambda i: (0,) * len(shape))


def _col1(i):
    return (i, 0)


def _tap_indices(Lat, Lon):
    i = Lat - 1
    j = Lon - 1
    fy = (i.astype(jnp.float32) + 0.5) / FACTOR - 0.5
    fx = (j.astype(jnp.float32) + 0.5) / FACTOR - 0.5
    y0 = jnp.floor(fy).astype(jnp.int32)
    x0 = jnp.floor(fx).astype(jnp.int32)
    ty = fy - y0.astype(jnp.float32)
    tx = fx - x0.astype(jnp.float32)
    y0c = jnp.clip(y0, 0, H - 2)
    y1c = jnp.clip(y0 + 1, 0, H - 2)
    x0c = jnp.clip(x0, 0, Wd - 1)
    x1c = jnp.clip(x0 + 1, 0, Wd - 1)
    pole = i == (H - 1) * FACTOR
    pole_idx = (H - 1) * Wd
    idx = jnp.stack([
        jnp.where(pole, pole_idx, y0c * Wd + x0c),
        jnp.where(pole, pole_idx, y0c * Wd + x1c),
        jnp.where(pole, pole_idx, y1c * Wd + x0c),
        jnp.where(pole, pole_idx, y1c * Wd + x1c),
    ])
    one = jnp.ones_like(ty)
    zero = jnp.zeros_like(ty)
    w4 = jnp.stack([
        jnp.where(pole, one, (1 - ty) * (1 - tx)),
        jnp.where(pole, zero, (1 - ty) * tx),
        jnp.where(pole, zero, ty * (1 - tx)),
        jnp.where(pole, zero, ty * tx),
    ], axis=1)
    return idx, w4


def kernel(x, edge_index, Lat, Lon, W1, b1, W2, b2, Lw, Lb, Lw1, Lb1):
    src = edge_index[0]
    dst = edge_index[1]

    xrow = jnp.pad(x[0].reshape(C, NG).T, ((0, 0), (0, D - C)))
    W1p = jnp.pad(W1, ((0, D - C), (0, 0)))
    b1r = b1.reshape(1, HID)
    b2r = b2.reshape(1, HID)
    lbr = Lb.reshape(1, HID)
    lb1r = Lb1.reshape(1, OUT_CH)
    idx4, w4 = _tap_indices(Lat, Lon)
    idxf = jnp.pad(idx4.T.reshape(-1), (0, MG - NMG))
    srcp = jnp.pad(src, (0, EP - src.shape[0]))
    dstp = jnp.pad(dst, (0, EP - dst.shape[0]), constant_values=-1)

    deg = _sc_deg(srcp, dstp)

    g = _sc_tap(idxf, xrow)
    gtap = g[:NMG].reshape(NM, 4 * D)

    lc, lbc = pl.pallas_call(
        _combine_dec_body,
        out_shape=(jax.ShapeDtypeStruct((HID, OUT_CH), jnp.float32),
                   jax.ShapeDtypeStruct((1, OUT_CH), jnp.float32)),
    )(Lw, Lw1, lbr, lb1r)

    grid_g, blk_g = _row_blocks(NG, D)
    xs_g, dis_g = pl.pallas_call(
        _scale_grid_body,
        grid=grid_g,
        in_specs=[blk_g, blk_g, blk_g],
        out_specs=(blk_g, pl.BlockSpec((BN, 1), _col1)),
        out_shape=(jax.ShapeDtypeStruct((NG, D), jnp.float32),
                   jax.ShapeDtypeStruct((NG, 1), jnp.float32)),
    )(xrow, deg[0, :NG], deg[1, :NG])

    grid_m, blk_m = _row_blocks(NM, D)
    xs_m, dis_m = pl.pallas_call(
        _mesh_assemble_body,
        grid=grid_m,
        in_specs=[pl.BlockSpec((BN, 4 * D), _col1),
                  pl.BlockSpec((BN, 4), _col1), blk_m, blk_m],
        out_specs=(blk_m, pl.BlockSpec((BN, 1), _col1)),
        out_shape=(jax.ShapeDtypeStruct((NM, D), jnp.float32),
                   jax.ShapeDtypeStruct((NM, 1), jnp.float32)),
    )(gtap, w4, deg[0, NG:NG + NM], deg[1, NG:NG + NM])

    xs = jnp.concatenate(
        [xs_g, xs_m, jnp.zeros((NPAD - N, D), jnp.float32)], axis=0)
    dis = jnp.concatenate([dis_g, dis_m], axis=0)

    (a1,) = _sc_agg1(srcp, dstp, xs)

    grid_n, blk_n = _row_blocks(N, D)
    gsa, gsb = pl.pallas_call(
        _gcn1_body,
        grid=grid_n,
        in_specs=[blk_n, blk_n, blk_n, pl.BlockSpec((BN, 1), _col1),
                  _full((D, HID)), _full((1, HID))],
        out_specs=(blk_n, blk_n),
        out_shape=(jax.ShapeDtypeStruct((N, D), jnp.float32),
                   jax.ShapeDtypeStruct((N, D), jnp.float32)),
    )(a1[0, :N], a1[1, :N], xs[:N], dis, W1p, b1r)

    pad_n = jnp.zeros((NPAD - N, D), jnp.float32)
    gsap = jnp.concatenate([gsa, pad_n], axis=0)
    gsbp = jnp.concatenate([gsb, pad_n], axis=0)

    a2a, a2b = _sc_agg2(srcp, dstp, gsap, gsbp)

    grid_d, blk_d = _row_blocks(NM, D)
    o = pl.pallas_call(
        _dec_body,
        grid=grid_d,
        in_specs=[blk_d, blk_d, blk_d, blk_d, blk_d, blk_d,
                  pl.BlockSpec((BN, 1), _col1),
                  _full((HID, HID)), _full((1, HID)),
                  _full((HID, OUT_CH)), _full((1, OUT_CH))],
        out_specs=pl.BlockSpec((BN, OUT_CH), _col1),
        out_shape=jax.ShapeDtypeStruct((NM, OUT_CH), jnp.float32),
    )(a2a[0, :NM], a2a[1, :NM], a2b[0, :NM], a2b[1, :NM],
      gsa[NG:], gsb[NG:], dis_m, W2, b2r, lc, lbc)

    return o.T[None]


# speedup vs baseline: 1.2132x; 1.2132x over previous
import math

import jax
import jax.numpy as jnp
from jax import lax
from jax.experimental import pallas as pl
from jax.experimental.pallas import tpu as pltpu
from jax.experimental.pallas import tpu_sc as plsc

C = 69
H = 121
Wd = 240
NG = H * Wd
NM = 40962
N = NG + NM
HID = 256
OUT_CH = 69
FACTOR = 4
D = 128
NPAD = 70016
BN = 512

NC, NS, LANES = 2, 16, 16
NW = NC * NS
EP = 303104
EPT = EP // NW
ECH = EPT // 128
MGQ = 5376
MG = NW * MGQ
NMG = 4 * NM

R1_NPASS, R1_SIZE = 5, 7552
R2_NPASS, R2_SIZE = 3, 6912

_SQRT2 = math.sqrt(2.0)
_sc_mesh = plsc.VectorSubcoreMesh(core_axis_name="c", subcore_axis_name="s",
                                  num_cores=NC, num_subcores=NS)


def _gelu(x):
    return x * 0.5 * (1.0 + jax.lax.erf(x / _SQRT2))


def _fill_zero(buf, nrows):
    z16 = jnp.zeros((LANES,), jnp.float32)

    def zr(i, _):
        for j in range(D // LANES):
            buf[i, pl.ds(j * LANES, LANES)] = z16
        return 0
    lax.fori_loop(0, nrows, zr, 0)


def _sc_tap_body(idxf_ref, xrow_ref, g_ref,
                 idx0, idx1, rows0, rows1, g0, g1, w0, w1):
    c = lax.axis_index("c")
    s = lax.axis_index("s")
    wid = s * NC + c
    npair = MGQ // 256

    def mg(u, _):
        off0 = wid * MGQ + (2 * u) * 128
        off1 = off0 + 128

        @pl.when(u > 0)
        def _wait_prev():
            po0 = off0 - 256
            pltpu.make_async_copy(rows0, g_ref.at[pl.ds(po0, 128)], w0).wait()
            pltpu.make_async_copy(rows1, g_ref.at[pl.ds(po0 + 128, 128)], w1).wait()

        pltpu.sync_copy(idxf_ref.at[pl.ds(off0, 128)], idx0.at[0])
        pltpu.sync_copy(idxf_ref.at[pl.ds(off1, 128)], idx1.at[0])
        cg0 = pltpu.async_copy(xrow_ref.at[idx0.at[0]], rows0, g0)
        cg1 = pltpu.async_copy(xrow_ref.at[idx1.at[0]], rows1, g1)
        cg0.wait()
        pltpu.async_copy(rows0, g_ref.at[pl.ds(off0, 128)], w0)
        cg1.wait()
        pltpu.async_copy(rows1, g_ref.at[pl.ds(off1, 128)], w1)
        return 0
    lax.fori_loop(0, npair, mg, 0)
    lo = wid * MGQ + (npair - 1) * 256
    pltpu.make_async_copy(rows0, g_ref.at[pl.ds(lo, 128)], w0).wait()
    pltpu.make_async_copy(rows1, g_ref.at[pl.ds(lo + 128, 128)], w1).wait()


_sc_tap = pl.kernel(
    _sc_tap_body,
    out_type=jax.ShapeDtypeStruct((MG, D), jnp.float32),
    mesh=_sc_mesh,
    scratch_types=[
        pltpu.VMEM((1, 128), jnp.int32),
        pltpu.VMEM((1, 128), jnp.int32),
        pltpu.VMEM((128, D), jnp.float32),
        pltpu.VMEM((128, D), jnp.float32),
        pltpu.SemaphoreType.DMA,
        pltpu.SemaphoreType.DMA,
        pltpu.SemaphoreType.DMA,
        pltpu.SemaphoreType.DMA,
    ],
)


def _make_agg(ntab, npass, rsize, base):
    rpt = rsize // NS
    acc_rows = rsize + 256

    def body(*refs):
        srcp_ref, dstp_ref = refs[0], refs[1]
        tabs = refs[2:2 + ntab]
        outs = refs[2 + ntab:2 + 2 * ntab] if ntab else (refs[2],)
        sc = refs[2 + 2 * ntab:] if ntab else refs[3:]
        (srcbuf, dstbuf, sidx0, sidx1, didx0, didx1,
         rows0, rows1, zbuf, acc, sg0, sg1, ss0, ss1) = sc

        c = lax.axis_index("c")
        s = lax.axis_index("s")
        wid = s * NC + c
        pltpu.sync_copy(srcp_ref.at[pl.ds(wid * EPT, EPT)], srcbuf)
        pltpu.sync_copy(dstp_ref.at[pl.ds(wid * EPT, EPT)], dstbuf)
        _fill_zero(zbuf, 128)
        if not ntab:
            one16 = jnp.ones((LANES,), jnp.float32)

            def fr(i, _):
                for j in range(D // LANES):
                    rows0[i, pl.ds(j * LANES, LANES)] = one16
                    rows1[i, pl.ds(j * LANES, LANES)] = one16
                return 0
            lax.fori_loop(0, 128, fr, 0)

        nz = acc_rows // 128
        nmy = (nz - s + NS - 1) // NS

        iot = lax.iota(jnp.int32, LANES)

        def build_idx(t, sidx, didx, lo, dof):
            for j in range(128 // LANES):
                d = dstbuf[pl.ds(t * 128 + j * LANES, LANES)]
                ld = d - lo
                inb = (ld >= 0) & (ld < rsize)
                dummy = rsize + dof + j * LANES + iot
                didx[0, pl.ds(j * LANES, LANES)] = jnp.where(inb, ld, dummy)
                if ntab:
                    sv = srcbuf[pl.ds(t * 128 + j * LANES, LANES)]
                    sidx[0, pl.ds(j * LANES, LANES)] = sv

        for h in range(max(ntab, 1)):
            out_ref = outs[h]
            for r in range(NC * npass):
                lo = base + r * rsize

                def za(i, _):
                    k = s + i * NS
                    pltpu.sync_copy(zbuf, acc.at[pl.ds(k * 128, 128)])
                    return 0
                lax.fori_loop(0, nmy, za, 0)
                plsc.subcore_barrier()

                def pair(u, _):
                    @pl.when(u > 0)
                    def _wait_prev():
                        pltpu.make_async_copy(rows0, acc.at[didx0.at[0]], ss0).wait()
                        pltpu.make_async_copy(rows1, acc.at[didx1.at[0]], ss1).wait()

                    build_idx(2 * u, sidx0, didx0, lo, 0)
                    build_idx(2 * u + 1, sidx1, didx1, lo, 128)
                    if ntab:
                        cg0 = pltpu.async_copy(tabs[h].at[sidx0.at[0]], rows0, sg0)
                        cg1 = pltpu.async_copy(tabs[h].at[sidx1.at[0]], rows1, sg1)
                        cg0.wait()
                        pltpu.async_copy(rows0, acc.at[didx0.at[0]], ss0, add=True)
                        cg1.wait()
                        pltpu.async_copy(rows1, acc.at[didx1.at[0]], ss1, add=True)
                    else:
                        pltpu.async_copy(rows0, acc.at[didx0.at[0]], ss0, add=True)
                        pltpu.async_copy(rows1, acc.at[didx1.at[0]], ss1, add=True)
                    return 0
                lax.fori_loop(0, ECH // 2, pair, 0)
                pltpu.make_async_copy(rows0, acc.at[didx0.at[0]], ss0).wait()
                pltpu.make_async_copy(rows1, acc.at[didx1.at[0]], ss1).wait()

                plsc.subcore_barrier()
                pltpu.sync_copy(acc.at[pl.ds(s * rpt, rpt)],
                                out_ref.at[c, pl.ds(r * rsize + s * rpt, rpt)])
                plsc.subcore_barrier()

    out_rows = NC * npass * rsize
    out1 = jax.ShapeDtypeStruct((NC, out_rows, D), jnp.float32)
    return pl.kernel(
        body,
        out_type=tuple([out1] * ntab) if ntab else out1,
        mesh=_sc_mesh,
        scratch_types=[
            pltpu.VMEM((EPT,), jnp.int32),
            pltpu.VMEM((EPT,), jnp.int32),
            pltpu.VMEM((1, 128), jnp.int32),
            pltpu.VMEM((1, 128), jnp.int32),
            pltpu.VMEM((1, 128), jnp.int32),
            pltpu.VMEM((1, 128), jnp.int32),
            pltpu.VMEM((128, D), jnp.float32),
            pltpu.VMEM((128, D), jnp.float32),
            pltpu.VMEM((128, D), jnp.float32),
            pltpu.VMEM_SHARED((acc_rows, D), jnp.float32),
            pltpu.SemaphoreType.DMA,
            pltpu.SemaphoreType.DMA,
            pltpu.SemaphoreType.DMA,
            pltpu.SemaphoreType.DMA,
        ],
    )


_sc_deg = _make_agg(0, R1_NPASS, R1_SIZE, 0)
_sc_agg1 = _make_agg(1, R1_NPASS, R1_SIZE, 0)
_sc_agg2 = _make_agg(2, R2_NPASS, R2_SIZE, NG)


def _combine_dec_body(lw_ref, lw1_ref, lb_ref, lb1_ref, lc_ref, lbc_ref):
    lc_ref[...] = jnp.dot(lw_ref[...], lw1_ref[...],
                          preferred_element_type=jnp.float32)
    lbc_ref[...] = jnp.dot(lb_ref[...], lw1_ref[...],
                           preferred_element_type=jnp.float32) + lb1_ref[...]


def _scale_grid_body(xrow_ref, deg0_ref, deg1_ref, xs_ref, dis_ref):
    dis = jax.lax.rsqrt(deg0_ref[...][:, :1] + deg1_ref[...][:, :1] + 1.0)
    dis_ref[...] = dis
    xs_ref[...] = xrow_ref[...] * dis


def _mesh_assemble_body(g_ref, w4_ref, deg0_ref, deg1_ref, xs_ref, dis_ref):
    g = g_ref[...]
    w = w4_ref[...]
    mesh = (g[:, 0 * D:1 * D] * w[:, 0:1] + g[:, 1 * D:2 * D] * w[:, 1:2]
            + g[:, 2 * D:3 * D] * w[:, 2:3] + g[:, 3 * D:4 * D] * w[:, 3:4])
    dis = jax.lax.rsqrt(deg0_ref[...][:, :1] + deg1_ref[...][:, :1] + 1.0)
    dis_ref[...] = dis
    xs_ref[...] = mesh * dis


def _gcn1_body(a10_ref, a11_ref, xs_ref, dis_ref, w1_ref, b1_ref, ga_ref, gb_ref):
    out1 = dis_ref[...] * (a10_ref[...] + a11_ref[...] + xs_ref[...])
    h = jnp.dot(out1, w1_ref[...], preferred_element_type=jnp.float32) + b1_ref[...]
    gs = _gelu(h) * dis_ref[...]
    ga_ref[...] = gs[:, :D]
    gb_ref[...] = gs[:, D:]


def _dec_body(a2a0_ref, a2a1_ref, a2b0_ref, a2b1_ref, gsa_ref, gsb_ref, dism_ref,
              w2_ref, b2_ref, lc_ref, lbc_ref, o_ref):
    dism = dism_ref[...]
    out2a = dism * (a2a0_ref[...] + a2a1_ref[...] + gsa_ref[...])
    out2b = dism * (a2b0_ref[...] + a2b1_ref[...] + gsb_ref[...])
    h2 = (jnp.dot(out2a, w2_ref[...][:D], preferred_element_type=jnp.float32)
          + jnp.dot(out2b, w2_ref[...][D:], preferred_element_type=jnp.float32)
          + b2_ref[...])
    o = jnp.dot(h2, lc_ref[...], preferred_element_type=jnp.float32) + lbc_ref[...]
    o_ref[...] = _gelu(o)


def _row_blocks(n_rows, width):
    grid = (pl.cdiv(n_rows, BN),)
    blk = pl.BlockSpec((BN, width), lambda i: (i, 0))
    return grid, blk


def _full(shape):
    return pl.BlockSpec(shape, lambda i: (0,) * len(shape))


def _col1(i):
    return (i, 0)


def _tap_indices(Lat, Lon):
    i = Lat - 1
    j = Lon - 1
    fy = (i.astype(jnp.float32) + 0.5) / FACTOR - 0.5
    fx = (j.astype(jnp.float32) + 0.5) / FACTOR - 0.5
    y0 = jnp.floor(fy).astype(jnp.int32)
    x0 = jnp.floor(fx).astype(jnp.int32)
    ty = fy - y0.astype(jnp.float32)
    tx = fx - x0.astype(jnp.float32)
    y0c = jnp.clip(y0, 0, H - 2)
    y1c = jnp.clip(y0 + 1, 0, H - 2)
    x0c = jnp.clip(x0, 0, Wd - 1)
    x1c = jnp.clip(x0 + 1, 0, Wd - 1)
    pole = i == (H - 1) * FACTOR
    pole_idx = (H - 1) * Wd
    idx = jnp.stack([
        jnp.where(pole, pole_idx, y0c * Wd + x0c),
        jnp.where(pole, pole_idx, y0c * Wd + x1c),
        jnp.where(pole, pole_idx, y1c * Wd + x0c),
        jnp.where(pole, pole_idx, y1c * Wd + x1c),
    ])
    one = jnp.ones_like(ty)
    zero = jnp.zeros_like(ty)
    w4 = jnp.stack([
        jnp.where(pole, one, (1 - ty) * (1 - tx)),
        jnp.where(pole, zero, (1 - ty) * tx),
        jnp.where(pole, zero, ty * (1 - tx)),
        jnp.where(pole, zero, ty * tx),
    ], axis=1)
    return idx, w4


def kernel(x, edge_index, Lat, Lon, W1, b1, W2, b2, Lw, Lb, Lw1, Lb1):
    src = edge_index[0]
    dst = edge_index[1]

    xrow = jnp.pad(x[0].reshape(C, NG).T, ((0, 0), (0, D - C)))
    W1p = jnp.pad(W1, ((0, D - C), (0, 0)))
    b1r = b1.reshape(1, HID)
    b2r = b2.reshape(1, HID)
    lbr = Lb.reshape(1, HID)
    lb1r = Lb1.reshape(1, OUT_CH)
    idx4, w4 = _tap_indices(Lat, Lon)
    idxf = jnp.pad(idx4.T.reshape(-1), (0, MG - NMG))
    srcp = jnp.pad(src, (0, EP - src.shape[0]))
    dstp = jnp.pad(dst, (0, EP - dst.shape[0]), constant_values=-1)

    deg = _sc_deg(srcp, dstp)

    g = _sc_tap(idxf, xrow)
    gtap = g[:NMG].reshape(NM, 4 * D)

    lc, lbc = pl.pallas_call(
        _combine_dec_body,
        out_shape=(jax.ShapeDtypeStruct((HID, OUT_CH), jnp.float32),
                   jax.ShapeDtypeStruct((1, OUT_CH), jnp.float32)),
    )(Lw, Lw1, lbr, lb1r)

    grid_g, blk_g = _row_blocks(NG, D)
    xs_g, dis_g = pl.pallas_call(
        _scale_grid_body,
        grid=grid_g,
        in_specs=[blk_g, blk_g, blk_g],
        out_specs=(blk_g, pl.BlockSpec((BN, 1), _col1)),
        out_shape=(jax.ShapeDtypeStruct((NG, D), jnp.float32),
                   jax.ShapeDtypeStruct((NG, 1), jnp.float32)),
    )(xrow, deg[0, :NG], deg[1, :NG])

    grid_m, blk_m = _row_blocks(NM, D)
    xs_m, dis_m = pl.pallas_call(
        _mesh_assemble_body,
        grid=grid_m,
        in_specs=[pl.BlockSpec((BN, 4 * D), _col1),
                  pl.BlockSpec((BN, 4), _col1), blk_m, blk_m],
        out_specs=(blk_m, pl.BlockSpec((BN, 1), _col1)),
        out_shape=(jax.ShapeDtypeStruct((NM, D), jnp.float32),
                   jax.ShapeDtypeStruct((NM, 1), jnp.float32)),
    )(gtap, w4, deg[0, NG:NG + NM], deg[1, NG:NG + NM])

    xs = jnp.concatenate(
        [xs_g, xs_m, jnp.zeros((NPAD - N, D), jnp.float32)], axis=0)
    dis = jnp.concatenate([dis_g, dis_m], axis=0)

    (a1,) = _sc_agg1(srcp, dstp, xs)

    grid_n, blk_n = _row_blocks(N, D)
    gsa, gsb = pl.pallas_call(
        _gcn1_body,
        grid=grid_n,
        in_specs=[blk_n, blk_n, blk_n, pl.BlockSpec((BN, 1), _col1),
                  _full((D, HID)), _full((1, HID))],
        out_specs=(blk_n, blk_n),
        out_shape=(jax.ShapeDtypeStruct((N, D), jnp.float32),
                   jax.ShapeDtypeStruct((N, D), jnp.float32)),
    )(a1[0, :N], a1[1, :N], xs[:N], dis, W1p, b1r)

    pad_n = jnp.zeros((NPAD - N, D), jnp.float32)
    gsap = jnp.concatenate([gsa, pad_n], axis=0)
    gsbp = jnp.concatenate([gsb, pad_n], axis=0)

    a2a, a2b = _sc_agg2(srcp, dstp, gsap, gsbp)

    grid_d, blk_d = _row_blocks(NM, D)
    o = pl.pallas_call(
        _dec_body,
        grid=grid_d,
        in_specs=[blk_d, blk_d, blk_d, blk_d, blk_d, blk_d,
                  pl.BlockSpec((BN, 1), _col1),
                  _full((HID, HID)), _full((1, HID)),
                  _full((HID, OUT_CH)), _full((1, OUT_CH))],
        out_specs=pl.BlockSpec((BN, OUT_CH), _col1),
        out_shape=jax.ShapeDtypeStruct((NM, OUT_CH), jnp.float32),
    )(a2a[0, :NM], a2a[1, :NM], a2b[0, :NM], a2b[1, :NM],
      gsa[NG:], gsb[NG:], dis_m, W2, b2r, lc, lbc)

    return o.T[None]
